# speedup vs baseline: 6.5838x; 6.5838x over previous
import functools

import jax
import jax.numpy as jnp
from jax.experimental import pallas as pl
from jax.experimental.pallas import tpu as pltpu

_NUM_FRAMES = 128
_CORES = 2
_TILE_N = 2048


def _round_up(x: int, m: int) -> int:
    return ((x + m - 1) // m) * m


def _seg_matmul_kernel(b_ref, x_ref, o_ref, *, num_frames):
    t = pl.program_id(1)

    @pl.when(t == 0)
    def _init():
        o_ref[...] = jnp.zeros_like(o_ref)

    b = b_ref[...]
    x = x_ref[...]
    fr = jax.lax.broadcasted_iota(jnp.int32, (b.shape[0], num_frames), 1)
    oh = (b == fr).astype(jnp.float32)
    part = jax.lax.dot_general(oh, x, (((0,), (0,)), ((), ())),
                               preferred_element_type=jnp.float32)
    o_ref[0] += part


def kernel(field, batch):
    field = jnp.asarray(field)
    n, d = field.shape
    num_frames = _NUM_FRAMES

    n_pad = _round_up(n, _CORES * _TILE_N)
    x = jnp.pad(field, ((0, n_pad - n), (0, 0)))
    b = jnp.pad(jnp.asarray(batch, jnp.int32), (0, n_pad - n),
                constant_values=num_frames).reshape(n_pad, 1)

    tiles = n_pad // (_CORES * _TILE_N)
    partials = pl.pallas_call(
        functools.partial(_seg_matmul_kernel, num_frames=num_frames),
        out_shape=jax.ShapeDtypeStruct((_CORES, num_frames, d), jnp.float32),
        grid=(_CORES, tiles),
        in_specs=[
            pl.BlockSpec((_TILE_N, 1), lambda c, t: (c * tiles + t, 0)),
            pl.BlockSpec((_TILE_N, d), lambda c, t: (c * tiles + t, 0)),
        ],
        out_specs=pl.BlockSpec((1, num_frames, d), lambda c, t: (c, 0, 0)),
        compiler_params=pltpu.CompilerParams(
            dimension_semantics=("parallel", "arbitrary"),
        ),
        cost_estimate=pl.CostEstimate(
            flops=2 * n_pad * num_frames * d,
            transcendentals=0,
            bytes_accessed=n_pad * (d * 4 + 4) + num_frames * d * 4,
        ),
    )(b, x)
    return jnp.sum(partials, axis=0)
